# baseline (speedup 1.0000x reference)
import functools

import jax
import jax.numpy as jnp
from jax import lax
from jax.experimental import pallas as pl
from jax.experimental.pallas import tpu as pltpu
from jax.experimental.pallas import tpu_sc as plsc

B = 16384
D = 64
NUM_USERS = 100000
NUM_AGE = 8
NUM_GENDER = 2
NUM_OCC = 21
NUM_GENRES = 19
NUM_COMBO = NUM_AGE * NUM_GENDER * NUM_OCC

_info = plsc.get_sparse_core_info()
NC, NS, L = _info.num_cores, _info.num_subcores, _info.num_lanes
BPT = B // NS
NV = D // L
P = D + 1
W = 192
UPT = 6256
NCH = (-(-UPT // W) + 1) // 2 * 2
HALF = BPT // 2
DQ = 128


def _sc_kernel():
    mesh = plsc.VectorSubcoreMesh(core_axis_name="c", subcore_axis_name="s")

    @functools.partial(
        pl.kernel,
        mesh=mesh,
        compiler_params=pltpu.CompilerParams(
            needs_layout_passes=False, use_tc_tiling_on_sc=False),
        out_type=[
            jax.ShapeDtypeStruct((B, D), jnp.float32),
            jax.ShapeDtypeStruct((B, D), jnp.float32),
            jax.ShapeDtypeStruct((B,), jnp.float32),
            jax.ShapeDtypeStruct((NUM_USERS, D), jnp.float32),
            jax.ShapeDtypeStruct((NUM_USERS, D), jnp.float32),
        ],
        scratch_types=[
            pltpu.VMEM((D, W), jnp.float32),
            pltpu.VMEM((D, W), jnp.float32),
            pltpu.VMEM((W, P), jnp.float32),
            pltpu.VMEM((HALF, D), jnp.float32),
            pltpu.VMEM((DQ, D), jnp.float32),
            pltpu.VMEM((BPT,), jnp.int32),
            pltpu.VMEM((BPT,), jnp.int32),
            pltpu.VMEM((BPT,), jnp.int32),
            pltpu.VMEM((BPT,), jnp.int32),
            pltpu.VMEM((BPT,), jnp.int32),
            pltpu.VMEM((BPT,), jnp.int32),
            pltpu.VMEM((HALF,), jnp.int32),
            pltpu.VMEM((DQ,), jnp.int32),
            pltpu.VMEM((BPT,), jnp.float32),
            pltpu.VMEM((BPT,), jnp.float32),
            pltpu.VMEM((BPT,), jnp.float32),
            pltpu.VMEM((NUM_AGE * D + NUM_GENDER * D + NUM_OCC * D,),
                       jnp.float32),
            pltpu.VMEM((NUM_OCC * D,), jnp.float32),
            pltpu.VMEM_SHARED((NUM_COMBO, D), jnp.float32),
            pltpu.SemaphoreType.DMA,
            pltpu.SemaphoreType.DMA,
            pltpu.SemaphoreType.DMA,
            pltpu.SemaphoreType.DMA,
            pltpu.SemaphoreType.DMA,
        ],
    )
    def sc_kernel(uidx_hbm, midx_hbm, aidx_hbm, gidx_hbm, oidx_hbm,
                  uembt_hbm, membt_hbm, tabs_hbm, ubias_hbm, mbias_hbm,
                  u_out, m_out, s_out, cu_out, cm_out,
                  stA, stB, tout_v, rows_v, drows_v,
                  uidx_v, midx_v, aidx_v, gidx_v, oidx_v, ci_v,
                  idxh_v, cib_v,
                  ub_v, mb_v, s_v, tabs_v, dstage_v, dtab_sh,
                  semA, semB, semO, semG, semD):
        cid = lax.axis_index("c")
        sid = lax.axis_index("s")
        ubase = sid * UPT
        sbase = sid * BPT

        riota = lax.iota(jnp.int32, L)

        def convert(src_t, dst):
            def cstart(c):
                return jnp.minimum(ubase + c * W, NUM_USERS - W)

            def in_cp(c, st, semI):
                return pltpu.make_async_copy(
                    src_t.at[pl.ds(0, D), pl.ds(cstart(c), W)], st, semI)

            in_cp(0, stA, semA).start()
            in_cp(1, stB, semB).start()

            def chunk_pair(i, carry):
                for b, (st, semI) in enumerate(((stA, semA), (stB, semB))):
                    c = i * 2 + b
                    in_cp(c, st, semI).wait()

                    @pl.when(c > 0)
                    def _():
                        pltpu.make_async_copy(
                            tout_v.at[pl.ds(0, W), pl.ds(0, D)],
                            dst.at[pl.ds(0, W)], semO).wait()

                    def tgroup(cg, carry2):
                        c0 = cg * L
                        rvec = c0 + riota
                        for d in range(D):
                            v = st[d, pl.ds(c0, L)]
                            plsc.store_scatter(
                                tout_v,
                                [rvec, jnp.full((L,), d, jnp.int32)], v)
                        return carry2
                    lax.fori_loop(0, W // L, tgroup, 0)
                    pltpu.async_copy(tout_v.at[pl.ds(0, W), pl.ds(0, D)],
                                     dst.at[pl.ds(cstart(c), W)], semO)

                    @pl.when(c + 2 < NCH)
                    def _():
                        in_cp(c + 2, st, semI).start()
                return carry
            lax.fori_loop(0, NCH // 2, chunk_pair, 0)
            pltpu.make_async_copy(tout_v.at[pl.ds(0, W), pl.ds(0, D)],
                                  dst.at[pl.ds(0, W)], semO).wait()

        @pl.when(cid == 0)
        def _():
            convert(uembt_hbm, cu_out)

        @pl.when(cid == 1)
        def _():
            convert(membt_hbm, cm_out)

        pltpu.sync_copy(uidx_hbm.at[pl.ds(sbase, BPT)], uidx_v)

        @pl.when(cid == 0)
        def _():
            pltpu.sync_copy(aidx_hbm.at[pl.ds(sbase, BPT)], aidx_v)
            pltpu.sync_copy(gidx_hbm.at[pl.ds(sbase, BPT)], gidx_v)
            pltpu.sync_copy(oidx_hbm.at[pl.ds(sbase, BPT)], oidx_v)
            pltpu.sync_copy(tabs_hbm, tabs_v)

            a_base = (sid // 2) * D
            g_base = NUM_AGE * D + (sid % 2) * D
            o_base = NUM_AGE * D + NUM_GENDER * D
            ag = [tabs_v[pl.ds(a_base + j * L, L)]
                  + tabs_v[pl.ds(g_base + j * L, L)] for j in range(NV)]
            for o in range(NUM_OCC):
                for j in range(NV):
                    dstage_v[pl.ds(o * D + j * L, L)] = (
                        ag[j] + tabs_v[pl.ds(o_base + o * D + j * L, L)])
            for o in range(NUM_OCC):
                pltpu.sync_copy(dstage_v.at[pl.ds(o * D, D)],
                                dtab_sh.at[sid * NUM_OCC + o])


        @pl.when(cid == 1)
        def _():
            pltpu.sync_copy(midx_hbm.at[pl.ds(sbase, BPT)], midx_v)
            cp_ub = pltpu.async_copy(ubias_hbm.at[uidx_v], ub_v, semG)
            cp_mb = pltpu.async_copy(mbias_hbm.at[midx_v], mb_v, semD)
            cp_ub.wait()
            cp_mb.wait()

            def s_body(g, carry):
                s_v[pl.ds(g * L, L)] = (
                    ub_v[pl.ds(g * L, L)] + mb_v[pl.ds(g * L, L)])
                return carry
            lax.fori_loop(0, BPT // L, s_body, 0)
            pltpu.sync_copy(s_v, s_out.at[pl.ds(sbase, BPT)])

        plsc.subcore_barrier()

        @pl.when(cid == 0)
        def _():
            for h in range(2):
                pltpu.sync_copy(uidx_hbm.at[pl.ds(sbase + h * HALF, HALF)],
                                idxh_v)
                cp_r = pltpu.async_copy(cu_out.at[idxh_v], rows_v, semG)

                def fill_ci(off):
                    def body(g, carry):
                        ai = aidx_v[pl.ds(off + g * L, L)]
                        gi = gidx_v[pl.ds(off + g * L, L)]
                        oi = oidx_v[pl.ds(off + g * L, L)]
                        cib_v[pl.ds(g * L, L)] = (
                            (ai * 2 + gi) * NUM_OCC + oi)
                        return carry
                    lax.fori_loop(0, DQ // L, body, 0)

                fill_ci(h * HALF)
                cp_d = pltpu.async_copy(dtab_sh.at[cib_v], drows_v, semD)
                cp_r.wait()
                for q in range(HALF // DQ):
                    cp_d.wait()

                    def add_body(g, carry):
                        r0 = q * DQ + g * L
                        for r in range(L):
                            for j in range(NV):
                                rows_v[r0 + r, pl.ds(j * L, L)] = (
                                    rows_v[r0 + r, pl.ds(j * L, L)]
                                    + drows_v[g * L + r, pl.ds(j * L, L)])
                        return carry
                    lax.fori_loop(0, DQ // L, add_body, 0)
                    if q + 1 < HALF // DQ:
                        fill_ci(h * HALF + (q + 1) * DQ)
                        cp_d = pltpu.async_copy(
                            dtab_sh.at[cib_v], drows_v, semD)
                pltpu.sync_copy(rows_v,
                                u_out.at[pl.ds(sbase + h * HALF, HALF)])

        @pl.when(cid == 1)
        def _():
            for h in range(2):
                pltpu.sync_copy(midx_hbm.at[pl.ds(sbase + h * HALF, HALF)],
                                idxh_v)
                cp_r = pltpu.async_copy(cm_out.at[idxh_v], rows_v, semG)
                cp_r.wait()
                pltpu.sync_copy(rows_v,
                                m_out.at[pl.ds(sbase + h * HALF, HALF)])

    return sc_kernel


_BK = 2048


def _tc_body(gb_ref, u_ref, m_ref, gv_ref, get_ref, s_ref, o_ref):
    u = u_ref[...]
    m = m_ref[...]
    t = jnp.dot(u, get_ref[...], preferred_element_type=jnp.float32)
    o_ref[...] = (s_ref[...] + gb_ref[0]
                  + jnp.sum(u * m, axis=1)
                  + jnp.sum(t * gv_ref[...], axis=1))


def kernel(user_idx, movie_idx, occ_idx, age_idx, gender_idx, genre_vec,
           user_emb, movie_emb, age_emb, gender_emb, occ_emb, genre_emb,
           user_bias, movie_bias, global_bias):
    user_idx = user_idx.astype(jnp.int32)
    movie_idx = movie_idx.astype(jnp.int32)
    occ_idx = occ_idx.astype(jnp.int32)
    age_idx = age_idx.astype(jnp.int32)
    gender_idx = gender_idx.astype(jnp.int32)

    tabs = jnp.concatenate(
        [age_emb.reshape(-1), gender_emb.reshape(-1), occ_emb.reshape(-1)])

    u_lat, m_rows, s_part, _, _ = _sc_kernel()(
        user_idx, movie_idx, age_idx, gender_idx, occ_idx,
        user_emb.T, movie_emb.T, tabs,
        user_bias.reshape(-1), movie_bias.reshape(-1))

    ge_t = genre_emb.T

    out = pl.pallas_call(
        _tc_body,
        grid=(B // _BK,),
        in_specs=[
            pl.BlockSpec(memory_space=pltpu.SMEM),
            pl.BlockSpec((_BK, D), lambda i: (i, 0)),
            pl.BlockSpec((_BK, D), lambda i: (i, 0)),
            pl.BlockSpec((_BK, NUM_GENRES), lambda i: (i, 0)),
            pl.BlockSpec((D, NUM_GENRES), lambda i: (0, 0)),
            pl.BlockSpec((_BK,), lambda i: (i,)),
        ],
        out_specs=pl.BlockSpec((_BK,), lambda i: (i,)),
        out_shape=jax.ShapeDtypeStruct((B,), jnp.float32),
    )(global_bias, u_lat, m_rows, genre_vec, ge_t, s_part)
    return out


# speedup vs baseline: 1.7054x; 1.7054x over previous
import functools

import jax
import jax.numpy as jnp
from jax import lax
from jax.experimental import pallas as pl
from jax.experimental.pallas import tpu as pltpu
from jax.experimental.pallas import tpu_sc as plsc

B = 16384
D = 64
NUM_AGE = 8
NUM_GENDER = 2
NUM_OCC = 21
NUM_GENRES = 19
NUM_COMBO = NUM_AGE * NUM_GENDER * NUM_OCC

_info = plsc.get_sparse_core_info()
NC, NS, L = _info.num_cores, _info.num_subcores, _info.num_lanes
NW = NC * NS
BPW = B // NW
NG = BPW // L
NV = D // L
PITCH = L + 1


def _sc_gather_kernel():
    mesh = plsc.VectorSubcoreMesh(core_axis_name="c", subcore_axis_name="s")

    @functools.partial(
        pl.kernel,
        mesh=mesh,
        compiler_params=pltpu.CompilerParams(
            needs_layout_passes=False, use_tc_tiling_on_sc=False),
        out_type=[
            jax.ShapeDtypeStruct((B, D), jnp.float32),
            jax.ShapeDtypeStruct((B,), jnp.float32),
        ],
        scratch_types=[
            pltpu.VMEM((BPW,), jnp.int32),
            pltpu.VMEM((BPW,), jnp.int32),
            pltpu.VMEM((BPW,), jnp.int32),
            pltpu.VMEM((BPW,), jnp.int32),
            pltpu.VMEM((BPW,), jnp.int32),
            pltpu.VMEM((BPW,), jnp.int32),
            pltpu.VMEM((BPW, D), jnp.float32),
            pltpu.VMEM((BPW, D), jnp.float32),
            pltpu.VMEM((BPW, D), jnp.float32),
            pltpu.VMEM((BPW,), jnp.float32),
            pltpu.VMEM((BPW,), jnp.float32),
            pltpu.VMEM((NUM_AGE * D + NUM_GENDER * D + NUM_OCC * D,),
                       jnp.float32),
            pltpu.VMEM((NUM_OCC * D,), jnp.float32),
            pltpu.VMEM((L * PITCH,), jnp.float32),
            pltpu.VMEM((BPW,), jnp.float32),
            pltpu.VMEM_SHARED((NUM_COMBO, D), jnp.float32),
            pltpu.SemaphoreType.DMA,
            pltpu.SemaphoreType.DMA,
            pltpu.SemaphoreType.DMA,
            pltpu.SemaphoreType.DMA,
            pltpu.SemaphoreType.DMA,
        ],
    )
    def sc_kernel(uidx_hbm, midx_hbm, aidx_hbm, gidx_hbm, oidx_hbm,
                  uemb_hbm, memb_hbm, tabs_hbm,
                  ubias_hbm, mbias_hbm,
                  u_out_hbm, s_out_hbm,
                  uidx_v, midx_v, aidx_v, gidx_v, oidx_v, ci_v,
                  urows_v, mrows_v, drows_v, ub_v, mb_v,
                  tabs_v, stage_v, ps_v, s_v, dtab_sh,
                  sem0, sem1, sem2, sem3, sem4):
        cid = lax.axis_index("c")
        sid = lax.axis_index("s")
        wid = sid * NC + cid
        base = wid * BPW

        pltpu.sync_copy(uidx_hbm.at[pl.ds(base, BPW)], uidx_v)
        pltpu.sync_copy(midx_hbm.at[pl.ds(base, BPW)], midx_v)
        cp_u = pltpu.async_copy(uemb_hbm.at[uidx_v], urows_v, sem0)
        cp_m = pltpu.async_copy(memb_hbm.at[midx_v], mrows_v, sem1)
        cp_ub = pltpu.async_copy(ubias_hbm.at[uidx_v], ub_v, sem2)
        cp_mb = pltpu.async_copy(mbias_hbm.at[midx_v], mb_v, sem3)

        pltpu.sync_copy(aidx_hbm.at[pl.ds(base, BPW)], aidx_v)
        pltpu.sync_copy(gidx_hbm.at[pl.ds(base, BPW)], gidx_v)
        pltpu.sync_copy(oidx_hbm.at[pl.ds(base, BPW)], oidx_v)
        pltpu.sync_copy(tabs_hbm, tabs_v)

        a_base = (sid // 2) * D
        g_base = NUM_AGE * D + (sid % 2) * D
        o_base = NUM_AGE * D + NUM_GENDER * D
        ag = [tabs_v[pl.ds(a_base + j * L, L)]
              + tabs_v[pl.ds(g_base + j * L, L)] for j in range(NV)]
        for o in range(NUM_OCC):
            for j in range(NV):
                stage_v[pl.ds(o * D + j * L, L)] = (
                    ag[j] + tabs_v[pl.ds(o_base + o * D + j * L, L)])
        for o in range(NUM_OCC):
            pltpu.sync_copy(stage_v.at[pl.ds(o * D, D)],
                            dtab_sh.at[sid * NUM_OCC + o])

        def ci_body(g, carry):
            ai = aidx_v[pl.ds(g * L, L)]
            gi = gidx_v[pl.ds(g * L, L)]
            oi = oidx_v[pl.ds(g * L, L)]
            ci_v[pl.ds(g * L, L)] = (ai * 2 + gi) * NUM_OCC + oi
            return carry
        lax.fori_loop(0, NG, ci_body, 0)

        plsc.subcore_barrier()
        cp_d = pltpu.async_copy(dtab_sh.at[ci_v], drows_v, sem4)

        cp_u.wait()
        cp_m.wait()
        cp_ub.wait()
        cp_mb.wait()
        cp_d.wait()

        cols = lax.iota(jnp.int32, L) * PITCH

        def group_body(g, carry):
            r0 = g * L
            for r in range(L):
                u = [urows_v[r0 + r, pl.ds(j * L, L)] for j in range(NV)]
                d = [drows_v[r0 + r, pl.ds(j * L, L)] for j in range(NV)]
                m = [mrows_v[r0 + r, pl.ds(j * L, L)] for j in range(NV)]
                p = None
                for j in range(NV):
                    uj = u[j] + d[j]
                    urows_v[r0 + r, pl.ds(j * L, L)] = uj
                    pj = uj * m[j]
                    p = pj if p is None else p + pj
                ps_v[pl.ds(r * PITCH, L)] = p
            acc = ub_v[pl.ds(r0, L)] + mb_v[pl.ds(r0, L)]
            for c in range(L):
                acc = acc + plsc.load_gather(ps_v, [cols + c])
            s_v[pl.ds(r0, L)] = acc
            return carry

        lax.fori_loop(0, NG, group_body, 0)

        pltpu.sync_copy(urows_v, u_out_hbm.at[pl.ds(base, BPW)])
        pltpu.sync_copy(s_v, s_out_hbm.at[pl.ds(base, BPW)])

    return sc_kernel


_BK = 2048


def _tc_body(gb_ref, u_ref, gv_ref, get_ref, s_ref, o_ref):
    t = jnp.dot(u_ref[...], get_ref[...], preferred_element_type=jnp.float32)
    o_ref[...] = s_ref[...] + gb_ref[0] + jnp.sum(t * gv_ref[...], axis=1)


def kernel(user_idx, movie_idx, occ_idx, age_idx, gender_idx, genre_vec,
           user_emb, movie_emb, age_emb, gender_emb, occ_emb, genre_emb,
           user_bias, movie_bias, global_bias):
    user_idx = user_idx.astype(jnp.int32)
    movie_idx = movie_idx.astype(jnp.int32)
    occ_idx = occ_idx.astype(jnp.int32)
    age_idx = age_idx.astype(jnp.int32)
    gender_idx = gender_idx.astype(jnp.int32)

    tabs = jnp.concatenate(
        [age_emb.reshape(-1), gender_emb.reshape(-1), occ_emb.reshape(-1)])

    u_lat, s_part = _sc_gather_kernel()(
        user_idx, movie_idx, age_idx, gender_idx, occ_idx,
        user_emb, movie_emb, tabs,
        user_bias.reshape(-1), movie_bias.reshape(-1))

    ge_t = genre_emb.T

    out = pl.pallas_call(
        _tc_body,
        grid=(B // _BK,),
        in_specs=[
            pl.BlockSpec(memory_space=pltpu.SMEM),
            pl.BlockSpec((_BK, D), lambda i: (i, 0)),
            pl.BlockSpec((_BK, NUM_GENRES), lambda i: (i, 0)),
            pl.BlockSpec((D, NUM_GENRES), lambda i: (0, 0)),
            pl.BlockSpec((_BK,), lambda i: (i,)),
        ],
        out_specs=pl.BlockSpec((_BK,), lambda i: (i,)),
        out_shape=jax.ShapeDtypeStruct((B,), jnp.float32),
    )(global_bias, u_lat, genre_vec, ge_t, s_part)
    return out
